# Initial kernel scaffold; baseline (speedup 1.0000x reference)
import functools

import jax
import jax.numpy as jnp
from jax import lax
from jax.experimental import pallas as pl
from jax.experimental.pallas import tpu as pltpu
from jax.experimental.pallas import tpu_sc as plsc

N = 10000
E = 320000
D_IN = 128
D = 256

NC = 2
NS = 16
LANES = 16

CHUNK = 80
EPT = E // NS
NCHT = EPT // CHUNK
EPW = E // (NC * NS)
ROWS_PT = N // NS
HALF = D // 2

_MESH = plsc.VectorSubcoreMesh(
    core_axis_name="c", subcore_axis_name="s", num_cores=NC, num_subcores=NS)


def _deg_body(eidx, out, dst_v, hist_v):
  c = lax.axis_index("c")
  s = lax.axis_index("s")
  wid = s * NC + c

  def zero_body(i, _):
    hist_v[pl.ds(i * LANES, LANES)] = jnp.zeros((LANES,), jnp.float32)
    return 0

  lax.fori_loop(0, N // LANES, zero_body, 0)
  pltpu.sync_copy(eidx.at[1, pl.ds(wid * EPW, EPW)], dst_v)

  ones = jnp.ones((LANES,), jnp.float32)

  def body(i, _):
    idx = dst_v[pl.ds(i * LANES, LANES)]
    plsc.addupdate_scatter(hist_v, [idx], ones)
    return 0

  lax.fori_loop(0, EPW // LANES, body, 0)
  pltpu.sync_copy(hist_v, out.at[wid])


_deg_kernel = functools.partial(
    pl.kernel,
    out_type=jax.ShapeDtypeStruct((NC * NS, N), jnp.float32),
    mesh=_MESH,
    scratch_types=[
        pltpu.VMEM((EPW,), jnp.int32),
        pltpu.VMEM((N,), jnp.float32),
    ],
)(_deg_body)


def _scatter_body(hp2, src_r, dst_r, out, src_v, dst_v, msg_v, zero_v, acc,
                  sem):
  c = lax.axis_index("c")
  s = lax.axis_index("s")

  def zb(i, _):
    r = i // (HALF // LANES)
    k = lax.rem(i, HALF // LANES)
    zero_v[r, pl.ds(k * LANES, LANES)] = jnp.zeros((LANES,), jnp.float32)
    return 0

  lax.fori_loop(0, 125 * (HALF // LANES), zb, 0)
  for k in range(ROWS_PT // 125):
    pltpu.sync_copy(zero_v, acc.at[pl.ds(s * ROWS_PT + k * 125, 125)])
  plsc.subcore_barrier()

  pltpu.sync_copy(src_r.at[pl.ds(s * NCHT, NCHT)], src_v)
  pltpu.sync_copy(dst_r.at[pl.ds(s * NCHT, NCHT)], dst_v)

  table = hp2.at[c]
  pltpu.make_async_copy(table.at[src_v.at[0]], msg_v.at[0], sem).start()

  def body(j, _):
    p = lax.rem(j, 2)
    pltpu.make_async_copy(table.at[src_v.at[j]], msg_v.at[p], sem).wait()
    nxt = j + 1

    @pl.when(nxt < NCHT)
    def _():
      pltpu.make_async_copy(
          table.at[src_v.at[nxt]], msg_v.at[lax.rem(nxt, 2)], sem).start()

    pltpu.sync_copy(msg_v.at[p], acc.at[dst_v.at[j]], add=True)
    return 0

  lax.fori_loop(0, NCHT, body, 0)
  plsc.subcore_barrier()
  pltpu.sync_copy(acc.at[pl.ds(s * ROWS_PT, ROWS_PT)],
                  out.at[c, pl.ds(s * ROWS_PT, ROWS_PT)])


_scatter_kernel = functools.partial(
    pl.kernel,
    out_type=jax.ShapeDtypeStruct((NC, N, HALF), jnp.float32),
    mesh=_MESH,
    scratch_types=[
        pltpu.VMEM((NCHT, CHUNK), jnp.int32),
        pltpu.VMEM((NCHT, CHUNK), jnp.int32),
        pltpu.VMEM((2, CHUNK, HALF), jnp.float32),
        pltpu.VMEM((125, HALF), jnp.float32),
        pltpu.VMEM_SHARED((N, HALF), jnp.float32),
        pltpu.SemaphoreType.DMA,
    ],
)(_scatter_body)


def _dinv_body(parts_ref, o_ref):
  ones = jnp.ones((NC * NS, 1), jnp.float32)
  deg = lax.dot_general(parts_ref[...], ones, (((0,), (0,)), ((), ())),
                        preferred_element_type=jnp.float32)
  o_ref[...] = lax.rsqrt(deg + 1.0)


def _tc_dinv(parts):
  return pl.pallas_call(
      _dinv_body,
      out_shape=jax.ShapeDtypeStruct((N, 1), jnp.float32),
  )(parts)


def _mm_body(x_ref, w_ref, dinv_ref, o_ref):
  h = jnp.dot(x_ref[...], w_ref[...], preferred_element_type=jnp.float32)
  h = h * dinv_ref[...]
  o_ref[0] = h[:, :HALF]
  o_ref[1] = h[:, HALF:]


def _tc_mm(x, w, dinv, bn=1000):
  din = x.shape[1]
  return pl.pallas_call(
      _mm_body,
      grid=(N // bn,),
      in_specs=[
          pl.BlockSpec((bn, din), lambda i: (i, 0)),
          pl.BlockSpec((din, D), lambda i: (0, 0)),
          pl.BlockSpec((bn, 1), lambda i: (i, 0)),
      ],
      out_specs=pl.BlockSpec((NC, bn, HALF), lambda i: (0, i, 0)),
      out_shape=jax.ShapeDtypeStruct((NC, N, HALF), jnp.float32),
  )(x, w, dinv)


def _post_body(s2_ref, hp2_ref, dinv_ref, b_ref, g_ref, be_ref, *rest):
  if len(rest) == 2:
    x_ref, o_ref = rest
  else:
    x_ref = None
    (o_ref,) = rest
  t = jnp.concatenate(
      [s2_ref[0] + hp2_ref[0], s2_ref[1] + hp2_ref[1]], axis=-1)
  t = t * dinv_ref[...] + b_ref[...]
  mu = jnp.mean(t, axis=-1, keepdims=True)
  d = t - mu
  var = jnp.mean(d * d, axis=-1, keepdims=True)
  y = d * lax.rsqrt(var + 1e-5) * g_ref[...] + be_ref[...]
  y = jnp.maximum(y, 0.0)
  if x_ref is not None:
    y = y + x_ref[...]
  o_ref[...] = y


def _final_body(s2_ref, hp2_ref, dinv_ref, b_ref, o_ref):
  t = jnp.concatenate(
      [s2_ref[0] + hp2_ref[0], s2_ref[1] + hp2_ref[1]], axis=-1)
  o_ref[...] = t * dinv_ref[...] + b_ref[...]


def _tc_post(s2, hp2, dinv, b, g, be, x_res=None, bn=1000):
  args = [s2, hp2, dinv, b, g, be]
  in_specs = [
      pl.BlockSpec((NC, bn, HALF), lambda i: (0, i, 0)),
      pl.BlockSpec((NC, bn, HALF), lambda i: (0, i, 0)),
      pl.BlockSpec((bn, 1), lambda i: (i, 0)),
      pl.BlockSpec((1, D), lambda i: (0, 0)),
      pl.BlockSpec((1, D), lambda i: (0, 0)),
      pl.BlockSpec((1, D), lambda i: (0, 0)),
  ]
  if x_res is not None:
    args.append(x_res)
    in_specs.append(pl.BlockSpec((bn, D), lambda i: (i, 0)))
  return pl.pallas_call(
      _post_body,
      grid=(N // bn,),
      in_specs=in_specs,
      out_specs=pl.BlockSpec((bn, D), lambda i: (i, 0)),
      out_shape=jax.ShapeDtypeStruct((N, D), jnp.float32),
  )(*args)


def _tc_final(s2, hp2, dinv, b, bn=1000):
  return pl.pallas_call(
      _final_body,
      grid=(N // bn,),
      in_specs=[
          pl.BlockSpec((NC, bn, HALF), lambda i: (0, i, 0)),
          pl.BlockSpec((NC, bn, HALF), lambda i: (0, i, 0)),
          pl.BlockSpec((bn, 1), lambda i: (i, 0)),
          pl.BlockSpec((1, D), lambda i: (0, 0)),
      ],
      out_specs=pl.BlockSpec((bn, D), lambda i: (i, 0)),
      out_shape=jax.ShapeDtypeStruct((N, D), jnp.float32),
  )(s2, hp2, dinv, b)


def kernel(features, edge_index, W0, b0, W1, b1, W2, b2, g0, be0, g1, be1):
  src_r = edge_index[0].reshape(E // CHUNK, CHUNK)
  dst_r = edge_index[1].reshape(E // CHUNK, CHUNK)
  b0r, b1r, b2r = b0.reshape(1, D), b1.reshape(1, D), b2.reshape(1, D)
  g0r, g1r = g0.reshape(1, D), g1.reshape(1, D)
  be0r, be1r = be0.reshape(1, D), be1.reshape(1, D)

  parts = _deg_kernel(edge_index)
  dinv = _tc_dinv(parts)

  hp2 = _tc_mm(features, W0, dinv)
  s2 = _scatter_kernel(hp2, src_r, dst_r)
  x = _tc_post(s2, hp2, dinv, b0r, g0r, be0r)

  hp2 = _tc_mm(x, W1, dinv)
  s2 = _scatter_kernel(hp2, src_r, dst_r)
  x = _tc_post(s2, hp2, dinv, b1r, g1r, be1r, x_res=x)

  hp2 = _tc_mm(x, W2, dinv)
  s2 = _scatter_kernel(hp2, src_r, dst_r)
  return _tc_final(s2, hp2, dinv, b2r)


# speedup vs baseline: 3.2022x; 3.2022x over previous
import functools

import jax
import jax.numpy as jnp
from jax import lax
from jax.experimental import pallas as pl
from jax.experimental.pallas import tpu as pltpu
from jax.experimental.pallas import tpu_sc as plsc

N = 10000
E = 320000
D_IN = 128
D = 256

NC = 2
NS = 16
LANES = 16

GCOLS = 4
NGRP = D // GCOLS
GPW = NGRP // (NC * NS)
ECH = 4000
NECH = E // ECH
EPW = E // (NC * NS)


@functools.cache
def _mesh():
  return plsc.VectorSubcoreMesh(
      core_axis_name="c", subcore_axis_name="s", num_cores=NC,
      num_subcores=NS)


def _deg_body(dst_flat, out, dst_v, hist_v):
  c = lax.axis_index("c")
  s = lax.axis_index("s")
  wid = s * NC + c

  def zero_body(i, _):
    hist_v[pl.ds(i * LANES, LANES)] = jnp.zeros((LANES,), jnp.float32)
    return 0

  lax.fori_loop(0, N // LANES, zero_body, 0)
  pltpu.sync_copy(dst_flat.at[pl.ds(wid * EPW, EPW)], dst_v)

  ones = jnp.ones((LANES,), jnp.float32)

  def body(i, _):
    idx = dst_v[pl.ds(i * LANES, LANES)]
    plsc.addupdate_scatter(hist_v, [idx], ones)
    return 0

  lax.fori_loop(0, EPW // LANES, body, 0)
  pltpu.sync_copy(hist_v, out.at[wid, 0])


@functools.cache
def _deg_kernel():
  return pl.kernel(
      _deg_body,
      out_type=jax.ShapeDtypeStruct((NC * NS, 1, N), jnp.float32),
      mesh=_mesh(),
      scratch_types=[
          pltpu.VMEM((EPW,), jnp.int32),
          pltpu.VMEM((N,), jnp.float32),
      ],
      compiler_params=pltpu.CompilerParams(needs_layout_passes=False),
  )


def _agg_body(hpt, src_r, dst_r, out, table_v, acc_v, src_v, dst_v, sem):
  c = lax.axis_index("c")
  s = lax.axis_index("s")
  widx = c * NS + s

  coff = [jnp.full((LANES,), ci * N, jnp.int32) for ci in range(GCOLS)]

  for g in range(GPW):
    gb = g * (NC * NS) + widx
    pltpu.sync_copy(hpt.at[gb, 0], table_v)

    def zero_body(i, _):
      acc_v[pl.ds(i * LANES, LANES)] = jnp.zeros((LANES,), jnp.float32)
      return 0

    lax.fori_loop(0, (GCOLS * N) // LANES, zero_body, 0)

    def idx_copy(ch, par):
      return (
          pltpu.make_async_copy(
              src_r.at[ch, 0], src_v.at[pl.ds(par * ECH, ECH)], sem),
          pltpu.make_async_copy(
              dst_r.at[ch, 0], dst_v.at[pl.ds(par * ECH, ECH)], sem),
      )

    for cp in idx_copy(0, 0):
      cp.start()

    def chunk_body(ch, _):
      par = lax.rem(ch, 2)
      for cp in idx_copy(ch, par):
        cp.wait()

      @pl.when(ch + 1 < NECH)
      def _():
        for cp in idx_copy(ch + 1, 1 - par):
          cp.start()

      base = par * ECH

      def vec_body(v, _):
        o = base + v * LANES
        sv = src_v[pl.ds(o, LANES)]
        dv = dst_v[pl.ds(o, LANES)]
        for ci in range(GCOLS):
          vals = plsc.load_gather(table_v, [sv + coff[ci]])
          plsc.addupdate_scatter(acc_v, [dv + coff[ci]], vals)
        return 0

      lax.fori_loop(0, ECH // LANES, vec_body, 0)
      return 0

    lax.fori_loop(0, NECH, chunk_body, 0)
    pltpu.sync_copy(acc_v, out.at[gb, 0])


@functools.cache
def _agg_kernel():
  return pl.kernel(
      _agg_body,
      out_type=jax.ShapeDtypeStruct((NGRP, 1, GCOLS * N), jnp.float32),
      mesh=_mesh(),
      scratch_types=[
          pltpu.VMEM((GCOLS * N,), jnp.float32),
          pltpu.VMEM((GCOLS * N,), jnp.float32),
          pltpu.VMEM((2 * ECH,), jnp.int32),
          pltpu.VMEM((2 * ECH,), jnp.int32),
          pltpu.SemaphoreType.DMA,
      ],
      compiler_params=pltpu.CompilerParams(needs_layout_passes=False),
  )


def _dinv_body(parts_ref, o_ref):
  parts = jnp.squeeze(parts_ref[...], axis=1)
  ones = jnp.ones((NC * NS, 1), jnp.float32)
  deg = lax.dot_general(parts, ones, (((0,), (0,)), ((), ())),
                        preferred_element_type=jnp.float32)
  o_ref[...] = lax.rsqrt(deg + 1.0)


def _tc_dinv(parts):
  return pl.pallas_call(
      _dinv_body,
      out_shape=jax.ShapeDtypeStruct((N, 1), jnp.float32),
  )(parts)


def _mmt_body(x_ref, w_ref, dinv_ref, o_ref):
  xs = x_ref[...] * dinv_ref[...]
  ht = lax.dot_general(w_ref[...], xs, (((0,), (1,)), ((), ())),
                       preferred_element_type=jnp.float32)
  o_ref[...] = ht.reshape(o_ref.shape)


def _tc_mmt(x, w, dinv):
  return pl.pallas_call(
      _mmt_body,
      out_shape=jax.ShapeDtypeStruct((NGRP, GCOLS, N), jnp.float32),
  )(x, w, dinv)


def _comb_body(st_ref, hpt_ref, dinv_ref, b_ref, o_ref):
  tt = st_ref[...].reshape(D, N) + hpt_ref[...].reshape(D, N)
  o_ref[...] = tt.T * dinv_ref[...] + b_ref[...]


def _tc_comb(st, hpt, dinv, b):
  return pl.pallas_call(
      _comb_body,
      out_shape=jax.ShapeDtypeStruct((N, D), jnp.float32),
  )(st, hpt, dinv, b)


def _ln_body(t_ref, g_ref, be_ref, fln_ref, fres_ref, x_ref, o_ref):
  t = t_ref[...]
  mu = jnp.mean(t, axis=-1, keepdims=True)
  d = t - mu
  var = jnp.mean(d * d, axis=-1, keepdims=True)
  y = d * lax.rsqrt(var + 1e-5) * g_ref[...] + be_ref[...]
  y = jnp.maximum(y, 0.0)
  fln = fln_ref[...]
  o_ref[...] = fln * y + (1.0 - fln) * t + fres_ref[...] * x_ref[...]


def _tc_ln(t, g, be, fln, fres, x_res, bn=2000):
  return pl.pallas_call(
      _ln_body,
      grid=(N // bn,),
      in_specs=[
          pl.BlockSpec((bn, D), lambda i: (i, 0)),
          pl.BlockSpec((1, D), lambda i: (0, 0)),
          pl.BlockSpec((1, D), lambda i: (0, 0)),
          pl.BlockSpec((1, 1), lambda i: (0, 0)),
          pl.BlockSpec((1, 1), lambda i: (0, 0)),
          pl.BlockSpec((bn, D), lambda i: (i, 0)),
      ],
      out_specs=pl.BlockSpec((bn, D), lambda i: (i, 0)),
      out_shape=jax.ShapeDtypeStruct((N, D), jnp.float32),
  )(t, g, be, fln, fres, x_res)


def kernel(features, edge_index, W0, b0, W1, b1, W2, b2, g0, be0, g1, be1):
  src_r = edge_index[0].reshape(NECH, 1, ECH)
  dst_r = edge_index[1].reshape(NECH, 1, ECH)
  dst_flat = edge_index[1]

  parts = _deg_kernel()(dst_flat)
  dinv = _tc_dinv(parts)

  x0 = jnp.pad(features, ((0, 0), (0, D - D_IN)))
  W0p = jnp.zeros((D, D), jnp.float32).at[:D_IN].set(W0)
  Ws = jnp.stack([W0p, W1, W2])
  bs = jnp.stack([b0, b1, b2]).reshape(3, 1, D)
  gs = jnp.stack([g0, g1, jnp.ones((D,), jnp.float32)]).reshape(3, 1, D)
  bes = jnp.stack([be0, be1, jnp.zeros((D,), jnp.float32)]).reshape(3, 1, D)
  flns = jnp.array([1.0, 1.0, 0.0], jnp.float32).reshape(3, 1, 1)
  fress = jnp.array([0.0, 1.0, 0.0], jnp.float32).reshape(3, 1, 1)

  def layer(x, params):
    w, b, g, be, fln, fres = params
    hpt = _tc_mmt(x, w, dinv)
    hpt_flat = hpt.reshape(NGRP, 1, GCOLS * N)
    st_flat = _agg_kernel()(hpt_flat, src_r, dst_r)
    st = st_flat.reshape(NGRP, GCOLS, N)
    t = _tc_comb(st, hpt, dinv, b)
    x_next = _tc_ln(t, g, be, fln, fres, x)
    return x_next, None

  out, _ = lax.scan(layer, x0, (Ws, bs, gs, bes, flns, fress))
  return out
